# Initial kernel scaffold; baseline (speedup 1.0000x reference)
import jax
import jax.numpy as jnp
from jax.experimental import pallas as pl


def kernel(inputs, embedding):
    raise NotImplementedError("write your pallas kernel here")


# speedup vs baseline: 6.4012x; 6.4012x over previous
import jax
import jax.numpy as jnp
from jax import lax
from jax.experimental import pallas as pl
from jax.experimental.pallas import tpu as pltpu
from jax.experimental.pallas import tpu_sc as plsc

_N = 8192
_K = 8192
_D = 32
_NBLK = 2048
_KBLK = 512

_NC, _NS, _L = 2, 16, 16
_NW = _NC * _NS
_BPW = _N // _NW
_CH = 128
_NCHK = _BPW // _CH
_SPW = _K // _NS


def _argmin_kernel(z_ref, e_ref, idx_ref, z2_ref, bval_ref, bidx_ref):
    k = pl.program_id(1)
    nk = pl.num_programs(1)

    @pl.when(k == 0)
    def _():
        z0 = z_ref[...]
        z2_ref[...] = jnp.sum(z0 * z0, axis=1, keepdims=True)

    z = z_ref[...]
    e = e_ref[...]
    mm = lax.dot_general(z, e, (((1,), (1,)), ((), ())),
                         preferred_element_type=jnp.float32)
    ones = jnp.ones((1, _D), jnp.float32)
    e2 = lax.dot_general(ones, e * e, (((1,), (1,)), ((), ())),
                         preferred_element_type=jnp.float32)
    scores = (z2_ref[...] + e2) - 2.0 * mm
    m = jnp.min(scores, axis=1, keepdims=True)
    col = lax.broadcasted_iota(jnp.int32, scores.shape, 1) + k * _KBLK
    amin = jnp.min(jnp.where(scores == m, col, _K), axis=1, keepdims=True)

    @pl.when(k == 0)
    def _():
        bval_ref[...] = m
        bidx_ref[...] = amin

    @pl.when(k > 0)
    def _():
        bv = bval_ref[...]
        upd = m < bv
        bval_ref[...] = jnp.where(upd, m, bv)
        bidx_ref[...] = jnp.where(upd, amin, bidx_ref[...])

    @pl.when(k == nk - 1)
    def _():
        idx_ref[...] = bidx_ref[...]


_argmin_call = pl.pallas_call(
    _argmin_kernel,
    grid=(_N // _NBLK, _K // _KBLK),
    in_specs=[
        pl.BlockSpec((_NBLK, _D), lambda n, k: (n, 0)),
        pl.BlockSpec((_KBLK, _D), lambda n, k: (k, 0)),
    ],
    out_specs=pl.BlockSpec((_NBLK, 1), lambda n, k: (n, 0)),
    out_shape=jax.ShapeDtypeStruct((_N, 1), jnp.int32),
    scratch_shapes=[
        pltpu.VMEM((_NBLK, 1), jnp.float32),
        pltpu.VMEM((_NBLK, 1), jnp.float32),
        pltpu.VMEM((_NBLK, 1), jnp.int32),
    ],
)


def _sc_body(emb_hbm, idx_hbm, q_hbm, cnt_hbm,
             idx_v, rows_v, ones_v, zer_v, cnt_sh, sem):
    c = lax.axis_index("c")
    s = lax.axis_index("s")
    wid = s * _NC + c
    base = wid * _BPW

    for i in range(_CH // _L):
        ones_v[pl.ds(i * _L, _L)] = jnp.ones((_L,), jnp.float32)
    for i in range(_SPW // _L):
        zer_v[pl.ds(i * _L, _L)] = jnp.zeros((_L,), jnp.float32)

    pltpu.sync_copy(zer_v, cnt_sh.at[pl.ds(s * _SPW, _SPW)])
    plsc.subcore_barrier()

    for j in range(_NCHK):
        pltpu.sync_copy(idx_hbm.at[pl.ds(base + j * _CH, _CH)], idx_v.at[j])
    for j in range(_NCHK):
        pltpu.async_copy(emb_hbm.at[idx_v.at[j]], rows_v.at[j], sem).wait()
        pltpu.sync_copy(rows_v.at[j], q_hbm.at[pl.ds(base + j * _CH, _CH)])
        pltpu.sync_copy(ones_v, cnt_sh.at[idx_v.at[j]], add=True)
    plsc.subcore_barrier()

    pltpu.sync_copy(cnt_sh.at[pl.ds(s * _SPW, _SPW)],
                    cnt_hbm.at[c, pl.ds(s * _SPW, _SPW)])


_sc_call = pl.kernel(
    _sc_body,
    out_type=(jax.ShapeDtypeStruct((_N, _D), jnp.float32),
              jax.ShapeDtypeStruct((_NC, _K), jnp.float32)),
    mesh=plsc.VectorSubcoreMesh(core_axis_name="c", subcore_axis_name="s"),
    compiler_params=pltpu.CompilerParams(use_tc_tiling_on_sc=False),
    scratch_types=[
        pltpu.VMEM((_NCHK, _CH), jnp.int32),
        pltpu.VMEM((_NCHK, _CH, _D), jnp.float32),
        pltpu.VMEM((_CH,), jnp.float32),
        pltpu.VMEM((_SPW,), jnp.float32),
        pltpu.VMEM_SHARED((_K,), jnp.float32),
        pltpu.SemaphoreType.DMA,
    ],
)


def _final_kernel(z_ref, q_ref, cnt_ref, qst_ref, loss_ref, perp_ref):
    z = z_ref[...]
    q = q_ref[...]
    d = q - z
    qst_ref[...] = z + d
    m = jnp.mean(d * d)
    loss_ref[...] = jnp.full((1, 1), m + 0.25 * m, jnp.float32)
    cnt = cnt_ref[0, :] + cnt_ref[1, :]
    p = cnt * (1.0 / _K)
    h = jnp.sum(p * jnp.log(p + 1e-10))
    perp_ref[...] = jnp.full((1, 1), jnp.exp(-h), jnp.float32)


_final_call = pl.pallas_call(
    _final_kernel,
    out_shape=(
        jax.ShapeDtypeStruct((_N, _D), jnp.float32),
        jax.ShapeDtypeStruct((1, 1), jnp.float32),
        jax.ShapeDtypeStruct((1, 1), jnp.float32),
    ),
)


def kernel(inputs, embedding):
    zf = inputs.reshape(_N, _D)
    idx = _argmin_call(zf, embedding).reshape(_N)
    q, counts = _sc_call(embedding, idx)
    qst, loss, perp = _final_call(zf, q, counts)
    return qst.reshape(inputs.shape), loss.reshape(()), perp.reshape(())


# speedup vs baseline: 7.5087x; 1.1730x over previous
import jax
import jax.numpy as jnp
from jax import lax
from jax.experimental import pallas as pl
from jax.experimental.pallas import tpu as pltpu
from jax.experimental.pallas import tpu_sc as plsc

_N = 8192
_K = 8192
_D = 32
_NBLK = 2048
_KBLK = 1024

_NC, _NS, _L = 2, 16, 16
_NW = _NC * _NS
_BPW = _N // _NW
_CH = 128
_NCHK = _BPW // _CH
_SPW = _K // _NS


def _argmin_kernel(z_ref, e_ref, idx_ref, z2_ref, bval_ref, bidx_ref):
    k = pl.program_id(1)
    nk = pl.num_programs(1)

    @pl.when(k == 0)
    def _():
        z0 = z_ref[...]
        z2_ref[...] = jnp.sum(z0 * z0, axis=1, keepdims=True)

    z = z_ref[...]
    e = e_ref[...]
    mm = lax.dot_general(z, e, (((1,), (1,)), ((), ())),
                         preferred_element_type=jnp.float32)
    ones = jnp.ones((1, _D), jnp.float32)
    e2 = lax.dot_general(ones, e * e, (((1,), (1,)), ((), ())),
                         preferred_element_type=jnp.float32)
    scores = (z2_ref[...] + e2) - 2.0 * mm
    m = jnp.min(scores, axis=1, keepdims=True)
    col = lax.broadcasted_iota(jnp.int32, scores.shape, 1) + k * _KBLK
    amin = jnp.min(jnp.where(scores == m, col, _K), axis=1, keepdims=True)

    @pl.when(k == 0)
    def _():
        bval_ref[...] = m
        bidx_ref[...] = amin

    @pl.when(k > 0)
    def _():
        bv = bval_ref[...]
        upd = m < bv
        bval_ref[...] = jnp.where(upd, m, bv)
        bidx_ref[...] = jnp.where(upd, amin, bidx_ref[...])

    @pl.when(k == nk - 1)
    def _():
        idx_ref[...] = bidx_ref[...]


_argmin_call = pl.pallas_call(
    _argmin_kernel,
    grid=(_N // _NBLK, _K // _KBLK),
    in_specs=[
        pl.BlockSpec((_NBLK, _D), lambda n, k: (n, 0)),
        pl.BlockSpec((_KBLK, _D), lambda n, k: (k, 0)),
    ],
    out_specs=pl.BlockSpec((_NBLK, 1), lambda n, k: (n, 0)),
    out_shape=jax.ShapeDtypeStruct((_N, 1), jnp.int32),
    scratch_shapes=[
        pltpu.VMEM((_NBLK, 1), jnp.float32),
        pltpu.VMEM((_NBLK, 1), jnp.float32),
        pltpu.VMEM((_NBLK, 1), jnp.int32),
    ],
)


def _sc_body(emb_hbm, idx_hbm, q_hbm, cnt_hbm,
             idx_v, rows_v, ones_v, zer_v, cnt_sh, sem):
    c = lax.axis_index("c")
    s = lax.axis_index("s")
    wid = s * _NC + c
    base = wid * _BPW

    for i in range(_CH // _L):
        ones_v[pl.ds(i * _L, _L)] = jnp.ones((_L,), jnp.float32)
    for i in range(_SPW // _L):
        zer_v[pl.ds(i * _L, _L)] = jnp.zeros((_L,), jnp.float32)

    pltpu.sync_copy(zer_v, cnt_sh.at[pl.ds(s * _SPW, _SPW)])
    plsc.subcore_barrier()

    for j in range(_NCHK):
        pltpu.sync_copy(idx_hbm.at[pl.ds(base + j * _CH, _CH)], idx_v.at[j])
    for j in range(_NCHK):
        pltpu.async_copy(emb_hbm.at[idx_v.at[j]], rows_v.at[j], sem).wait()
        pltpu.sync_copy(rows_v.at[j], q_hbm.at[pl.ds(base + j * _CH, _CH)])
        pltpu.sync_copy(ones_v, cnt_sh.at[idx_v.at[j]], add=True)
    plsc.subcore_barrier()

    pltpu.sync_copy(cnt_sh.at[pl.ds(s * _SPW, _SPW)],
                    cnt_hbm.at[c, pl.ds(s * _SPW, _SPW)])


_sc_call = pl.kernel(
    _sc_body,
    out_type=(jax.ShapeDtypeStruct((_N, _D), jnp.float32),
              jax.ShapeDtypeStruct((_NC, _K), jnp.float32)),
    mesh=plsc.VectorSubcoreMesh(core_axis_name="c", subcore_axis_name="s"),
    compiler_params=pltpu.CompilerParams(use_tc_tiling_on_sc=False),
    scratch_types=[
        pltpu.VMEM((_NCHK, _CH), jnp.int32),
        pltpu.VMEM((_NCHK, _CH, _D), jnp.float32),
        pltpu.VMEM((_CH,), jnp.float32),
        pltpu.VMEM((_SPW,), jnp.float32),
        pltpu.VMEM_SHARED((_K,), jnp.float32),
        pltpu.SemaphoreType.DMA,
    ],
)


def _final_kernel(z_ref, q_ref, cnt_ref, qst_ref, loss_ref, perp_ref):
    z = z_ref[...]
    q = q_ref[...]
    d = q - z
    qst_ref[...] = z + d
    m = jnp.mean(d * d)
    loss_ref[...] = jnp.full((1, 1), m + 0.25 * m, jnp.float32)
    cnt = cnt_ref[0, :] + cnt_ref[1, :]
    p = cnt * (1.0 / _K)
    h = jnp.sum(p * jnp.log(p + 1e-10))
    perp_ref[...] = jnp.full((1, 1), jnp.exp(-h), jnp.float32)


_final_call = pl.pallas_call(
    _final_kernel,
    out_shape=(
        jax.ShapeDtypeStruct((_N, _D), jnp.float32),
        jax.ShapeDtypeStruct((1, 1), jnp.float32),
        jax.ShapeDtypeStruct((1, 1), jnp.float32),
    ),
)


def kernel(inputs, embedding):
    zf = inputs.reshape(_N, _D)
    idx = _argmin_call(zf, embedding).reshape(_N)
    q, counts = _sc_call(embedding, idx)
    qst, loss, perp = _final_call(zf, q, counts)
    return qst.reshape(inputs.shape), loss.reshape(()), perp.reshape(())


# speedup vs baseline: 8.5221x; 1.1350x over previous
import jax
import jax.numpy as jnp
from jax import lax
from jax.experimental import pallas as pl
from jax.experimental.pallas import tpu as pltpu
from jax.experimental.pallas import tpu_sc as plsc

_N = 8192
_K = 8192
_D = 32
_NBLK = 2048
_KBLK = 2048

_NC, _NS, _L = 2, 16, 16
_NW = _NC * _NS
_BPW = _N // _NW
_CH = 128
_NCHK = _BPW // _CH
_SPW = _K // _NS


def _argmin_kernel(z_ref, e_ref, idx_ref, z2_ref, bval_ref, bidx_ref):
    k = pl.program_id(1)
    nk = pl.num_programs(1)

    @pl.when(k == 0)
    def _():
        z0 = z_ref[...]
        z2_ref[...] = jnp.sum(z0 * z0, axis=1, keepdims=True)

    z = z_ref[...]
    e = e_ref[...]
    mm = lax.dot_general(z, e, (((1,), (1,)), ((), ())),
                         preferred_element_type=jnp.float32)
    ones = jnp.ones((1, _D), jnp.float32)
    e2 = lax.dot_general(ones, e * e, (((1,), (1,)), ((), ())),
                         preferred_element_type=jnp.float32)
    scores = (z2_ref[...] + e2) - 2.0 * mm
    m = jnp.min(scores, axis=1, keepdims=True)
    col = lax.broadcasted_iota(jnp.int32, scores.shape, 1) + k * _KBLK
    amin = jnp.min(jnp.where(scores == m, col, _K), axis=1, keepdims=True)

    @pl.when(k == 0)
    def _():
        bval_ref[...] = m
        bidx_ref[...] = amin

    @pl.when(k > 0)
    def _():
        bv = bval_ref[...]
        upd = m < bv
        bval_ref[...] = jnp.where(upd, m, bv)
        bidx_ref[...] = jnp.where(upd, amin, bidx_ref[...])

    @pl.when(k == nk - 1)
    def _():
        idx_ref[...] = bidx_ref[...]


_argmin_call = pl.pallas_call(
    _argmin_kernel,
    grid=(_N // _NBLK, _K // _KBLK),
    in_specs=[
        pl.BlockSpec((_NBLK, _D), lambda n, k: (n, 0)),
        pl.BlockSpec((_KBLK, _D), lambda n, k: (k, 0)),
    ],
    out_specs=pl.BlockSpec((_NBLK, 1), lambda n, k: (n, 0)),
    out_shape=jax.ShapeDtypeStruct((_N, 1), jnp.int32),
    scratch_shapes=[
        pltpu.VMEM((_NBLK, 1), jnp.float32),
        pltpu.VMEM((_NBLK, 1), jnp.float32),
        pltpu.VMEM((_NBLK, 1), jnp.int32),
    ],
)


def _sc_body(emb_hbm, idx_hbm, q_hbm, cnt_hbm,
             idx_v, rows_v, ones_v, zer_v, cnt_sh, sem):
    c = lax.axis_index("c")
    s = lax.axis_index("s")
    wid = s * _NC + c
    base = wid * _BPW

    for i in range(_CH // _L):
        ones_v[pl.ds(i * _L, _L)] = jnp.ones((_L,), jnp.float32)
    for i in range(_SPW // _L):
        zer_v[pl.ds(i * _L, _L)] = jnp.zeros((_L,), jnp.float32)

    pltpu.sync_copy(zer_v, cnt_sh.at[pl.ds(s * _SPW, _SPW)])
    plsc.subcore_barrier()

    for j in range(_NCHK):
        pltpu.sync_copy(idx_hbm.at[pl.ds(base + j * _CH, _CH)], idx_v.at[j])
    for j in range(_NCHK):
        pltpu.async_copy(emb_hbm.at[idx_v.at[j]], rows_v.at[j], sem).wait()
        pltpu.sync_copy(rows_v.at[j], q_hbm.at[pl.ds(base + j * _CH, _CH)])
        pltpu.sync_copy(ones_v, cnt_sh.at[idx_v.at[j]], add=True)
    plsc.subcore_barrier()

    pltpu.sync_copy(cnt_sh.at[pl.ds(s * _SPW, _SPW)],
                    cnt_hbm.at[c, pl.ds(s * _SPW, _SPW)])


_sc_call = pl.kernel(
    _sc_body,
    out_type=(jax.ShapeDtypeStruct((_N, _D), jnp.float32),
              jax.ShapeDtypeStruct((_NC, _K), jnp.float32)),
    mesh=plsc.VectorSubcoreMesh(core_axis_name="c", subcore_axis_name="s"),
    compiler_params=pltpu.CompilerParams(use_tc_tiling_on_sc=False),
    scratch_types=[
        pltpu.VMEM((_NCHK, _CH), jnp.int32),
        pltpu.VMEM((_NCHK, _CH, _D), jnp.float32),
        pltpu.VMEM((_CH,), jnp.float32),
        pltpu.VMEM((_SPW,), jnp.float32),
        pltpu.VMEM_SHARED((_K,), jnp.float32),
        pltpu.SemaphoreType.DMA,
    ],
)


def _final_kernel(z_ref, q_ref, cnt_ref, qst_ref, loss_ref, perp_ref):
    z = z_ref[...]
    q = q_ref[...]
    d = q - z
    qst_ref[...] = z + d
    m = jnp.mean(d * d)
    loss_ref[...] = jnp.full((1, 1), m + 0.25 * m, jnp.float32)
    cnt = cnt_ref[0, :] + cnt_ref[1, :]
    p = cnt * (1.0 / _K)
    h = jnp.sum(p * jnp.log(p + 1e-10))
    perp_ref[...] = jnp.full((1, 1), jnp.exp(-h), jnp.float32)


_final_call = pl.pallas_call(
    _final_kernel,
    out_shape=(
        jax.ShapeDtypeStruct((_N, _D), jnp.float32),
        jax.ShapeDtypeStruct((1, 1), jnp.float32),
        jax.ShapeDtypeStruct((1, 1), jnp.float32),
    ),
)


def kernel(inputs, embedding):
    zf = inputs.reshape(_N, _D)
    idx = _argmin_call(zf, embedding).reshape(_N)
    q, counts = _sc_call(embedding, idx)
    qst, loss, perp = _final_call(zf, q, counts)
    return qst.reshape(inputs.shape), loss.reshape(()), perp.reshape(())


# speedup vs baseline: 9.3863x; 1.1014x over previous
import jax
import jax.numpy as jnp
from jax import lax
from jax.experimental import pallas as pl
from jax.experimental.pallas import tpu as pltpu
from jax.experimental.pallas import tpu_sc as plsc

_N = 8192
_K = 8192
_D = 32
_NBLK = 2048
_KBLK = 4096

_NC, _NS, _L = 2, 16, 16
_NW = _NC * _NS
_BPW = _N // _NW
_CH = 128
_NCHK = _BPW // _CH
_SPW = _K // _NS


def _argmin_kernel(z_ref, e_ref, idx_ref, z2_ref, bval_ref, bidx_ref):
    k = pl.program_id(1)
    nk = pl.num_programs(1)

    @pl.when(k == 0)
    def _():
        z0 = z_ref[...]
        z2_ref[...] = jnp.sum(z0 * z0, axis=1, keepdims=True)

    z = z_ref[...]
    e = e_ref[...]
    mm = lax.dot_general(z, e, (((1,), (1,)), ((), ())),
                         preferred_element_type=jnp.float32)
    ones = jnp.ones((1, _D), jnp.float32)
    e2 = lax.dot_general(ones, e * e, (((1,), (1,)), ((), ())),
                         preferred_element_type=jnp.float32)
    scores = (z2_ref[...] + e2) - 2.0 * mm
    m = jnp.min(scores, axis=1, keepdims=True)
    col = lax.broadcasted_iota(jnp.int32, scores.shape, 1) + k * _KBLK
    amin = jnp.min(jnp.where(scores == m, col, _K), axis=1, keepdims=True)

    @pl.when(k == 0)
    def _():
        bval_ref[...] = m
        bidx_ref[...] = amin

    @pl.when(k > 0)
    def _():
        bv = bval_ref[...]
        upd = m < bv
        bval_ref[...] = jnp.where(upd, m, bv)
        bidx_ref[...] = jnp.where(upd, amin, bidx_ref[...])

    @pl.when(k == nk - 1)
    def _():
        idx_ref[...] = bidx_ref[...]


_argmin_call = pl.pallas_call(
    _argmin_kernel,
    grid=(_N // _NBLK, _K // _KBLK),
    in_specs=[
        pl.BlockSpec((_NBLK, _D), lambda n, k: (n, 0)),
        pl.BlockSpec((_KBLK, _D), lambda n, k: (k, 0)),
    ],
    out_specs=pl.BlockSpec((_NBLK, 1), lambda n, k: (n, 0)),
    out_shape=jax.ShapeDtypeStruct((_N, 1), jnp.int32),
    scratch_shapes=[
        pltpu.VMEM((_NBLK, 1), jnp.float32),
        pltpu.VMEM((_NBLK, 1), jnp.float32),
        pltpu.VMEM((_NBLK, 1), jnp.int32),
    ],
)


def _sc_body(emb_hbm, idx_hbm, q_hbm, cnt_hbm,
             idx_v, rows_v, ones_v, zer_v, cnt_sh, sem):
    c = lax.axis_index("c")
    s = lax.axis_index("s")
    wid = s * _NC + c
    base = wid * _BPW

    for i in range(_CH // _L):
        ones_v[pl.ds(i * _L, _L)] = jnp.ones((_L,), jnp.float32)
    for i in range(_SPW // _L):
        zer_v[pl.ds(i * _L, _L)] = jnp.zeros((_L,), jnp.float32)

    pltpu.sync_copy(zer_v, cnt_sh.at[pl.ds(s * _SPW, _SPW)])
    plsc.subcore_barrier()

    for j in range(_NCHK):
        pltpu.sync_copy(idx_hbm.at[pl.ds(base + j * _CH, _CH)], idx_v.at[j])
    for j in range(_NCHK):
        pltpu.async_copy(emb_hbm.at[idx_v.at[j]], rows_v.at[j], sem).wait()
        pltpu.sync_copy(rows_v.at[j], q_hbm.at[pl.ds(base + j * _CH, _CH)])
        pltpu.sync_copy(ones_v, cnt_sh.at[idx_v.at[j]], add=True)
    plsc.subcore_barrier()

    pltpu.sync_copy(cnt_sh.at[pl.ds(s * _SPW, _SPW)],
                    cnt_hbm.at[c, pl.ds(s * _SPW, _SPW)])


_sc_call = pl.kernel(
    _sc_body,
    out_type=(jax.ShapeDtypeStruct((_N, _D), jnp.float32),
              jax.ShapeDtypeStruct((_NC, _K), jnp.float32)),
    mesh=plsc.VectorSubcoreMesh(core_axis_name="c", subcore_axis_name="s"),
    compiler_params=pltpu.CompilerParams(use_tc_tiling_on_sc=False),
    scratch_types=[
        pltpu.VMEM((_NCHK, _CH), jnp.int32),
        pltpu.VMEM((_NCHK, _CH, _D), jnp.float32),
        pltpu.VMEM((_CH,), jnp.float32),
        pltpu.VMEM((_SPW,), jnp.float32),
        pltpu.VMEM_SHARED((_K,), jnp.float32),
        pltpu.SemaphoreType.DMA,
    ],
)


def _final_kernel(z_ref, q_ref, cnt_ref, qst_ref, loss_ref, perp_ref):
    z = z_ref[...]
    q = q_ref[...]
    d = q - z
    qst_ref[...] = z + d
    m = jnp.mean(d * d)
    loss_ref[...] = jnp.full((1, 1), m + 0.25 * m, jnp.float32)
    cnt = cnt_ref[0, :] + cnt_ref[1, :]
    p = cnt * (1.0 / _K)
    h = jnp.sum(p * jnp.log(p + 1e-10))
    perp_ref[...] = jnp.full((1, 1), jnp.exp(-h), jnp.float32)


_final_call = pl.pallas_call(
    _final_kernel,
    out_shape=(
        jax.ShapeDtypeStruct((_N, _D), jnp.float32),
        jax.ShapeDtypeStruct((1, 1), jnp.float32),
        jax.ShapeDtypeStruct((1, 1), jnp.float32),
    ),
)


def kernel(inputs, embedding):
    zf = inputs.reshape(_N, _D)
    idx = _argmin_call(zf, embedding).reshape(_N)
    q, counts = _sc_call(embedding, idx)
    qst, loss, perp = _final_call(zf, q, counts)
    return qst.reshape(inputs.shape), loss.reshape(()), perp.reshape(())


# speedup vs baseline: 9.4945x; 1.0115x over previous
import jax
import jax.numpy as jnp
from jax import lax
from jax.experimental import pallas as pl
from jax.experimental.pallas import tpu as pltpu
from jax.experimental.pallas import tpu_sc as plsc

_N = 8192
_K = 8192
_D = 32
_NBLK = 1024
_KBLK = 8192

_NC, _NS, _L = 2, 16, 16
_NW = _NC * _NS
_BPW = _N // _NW
_CH = 128
_NCHK = _BPW // _CH
_SPW = _K // _NS


def _argmin_kernel(z_ref, e_ref, idx_ref, z2_ref, bval_ref, bidx_ref):
    k = pl.program_id(1)
    nk = pl.num_programs(1)

    @pl.when(k == 0)
    def _():
        z0 = z_ref[...]
        z2_ref[...] = jnp.sum(z0 * z0, axis=1, keepdims=True)

    z = z_ref[...]
    e = e_ref[...]
    mm = lax.dot_general(z, e, (((1,), (1,)), ((), ())),
                         preferred_element_type=jnp.float32)
    ones = jnp.ones((1, _D), jnp.float32)
    e2 = lax.dot_general(ones, e * e, (((1,), (1,)), ((), ())),
                         preferred_element_type=jnp.float32)
    scores = (z2_ref[...] + e2) - 2.0 * mm
    m = jnp.min(scores, axis=1, keepdims=True)
    col = lax.broadcasted_iota(jnp.int32, scores.shape, 1) + k * _KBLK
    amin = jnp.min(jnp.where(scores == m, col, _K), axis=1, keepdims=True)

    @pl.when(k == 0)
    def _():
        bval_ref[...] = m
        bidx_ref[...] = amin

    @pl.when(k > 0)
    def _():
        bv = bval_ref[...]
        upd = m < bv
        bval_ref[...] = jnp.where(upd, m, bv)
        bidx_ref[...] = jnp.where(upd, amin, bidx_ref[...])

    @pl.when(k == nk - 1)
    def _():
        idx_ref[...] = bidx_ref[...]


_argmin_call = pl.pallas_call(
    _argmin_kernel,
    grid=(_N // _NBLK, _K // _KBLK),
    in_specs=[
        pl.BlockSpec((_NBLK, _D), lambda n, k: (n, 0)),
        pl.BlockSpec((_KBLK, _D), lambda n, k: (k, 0)),
    ],
    out_specs=pl.BlockSpec((_NBLK, 1), lambda n, k: (n, 0)),
    out_shape=jax.ShapeDtypeStruct((_N, 1), jnp.int32),
    scratch_shapes=[
        pltpu.VMEM((_NBLK, 1), jnp.float32),
        pltpu.VMEM((_NBLK, 1), jnp.float32),
        pltpu.VMEM((_NBLK, 1), jnp.int32),
    ],
)


def _sc_body(emb_hbm, idx_hbm, q_hbm, cnt_hbm,
             idx_v, rows_v, ones_v, zer_v, cnt_sh, sem):
    c = lax.axis_index("c")
    s = lax.axis_index("s")
    wid = s * _NC + c
    base = wid * _BPW

    for i in range(_CH // _L):
        ones_v[pl.ds(i * _L, _L)] = jnp.ones((_L,), jnp.float32)
    for i in range(_SPW // _L):
        zer_v[pl.ds(i * _L, _L)] = jnp.zeros((_L,), jnp.float32)

    pltpu.sync_copy(zer_v, cnt_sh.at[pl.ds(s * _SPW, _SPW)])
    plsc.subcore_barrier()

    for j in range(_NCHK):
        pltpu.sync_copy(idx_hbm.at[pl.ds(base + j * _CH, _CH)], idx_v.at[j])
    for j in range(_NCHK):
        pltpu.async_copy(emb_hbm.at[idx_v.at[j]], rows_v.at[j], sem).wait()
        pltpu.sync_copy(rows_v.at[j], q_hbm.at[pl.ds(base + j * _CH, _CH)])
        pltpu.sync_copy(ones_v, cnt_sh.at[idx_v.at[j]], add=True)
    plsc.subcore_barrier()

    pltpu.sync_copy(cnt_sh.at[pl.ds(s * _SPW, _SPW)],
                    cnt_hbm.at[c, pl.ds(s * _SPW, _SPW)])


_sc_call = pl.kernel(
    _sc_body,
    out_type=(jax.ShapeDtypeStruct((_N, _D), jnp.float32),
              jax.ShapeDtypeStruct((_NC, _K), jnp.float32)),
    mesh=plsc.VectorSubcoreMesh(core_axis_name="c", subcore_axis_name="s"),
    compiler_params=pltpu.CompilerParams(use_tc_tiling_on_sc=False),
    scratch_types=[
        pltpu.VMEM((_NCHK, _CH), jnp.int32),
        pltpu.VMEM((_NCHK, _CH, _D), jnp.float32),
        pltpu.VMEM((_CH,), jnp.float32),
        pltpu.VMEM((_SPW,), jnp.float32),
        pltpu.VMEM_SHARED((_K,), jnp.float32),
        pltpu.SemaphoreType.DMA,
    ],
)


def _final_kernel(z_ref, q_ref, cnt_ref, qst_ref, loss_ref, perp_ref):
    z = z_ref[...]
    q = q_ref[...]
    d = q - z
    qst_ref[...] = z + d
    m = jnp.mean(d * d)
    loss_ref[...] = jnp.full((1, 1), m + 0.25 * m, jnp.float32)
    cnt = cnt_ref[0, :] + cnt_ref[1, :]
    p = cnt * (1.0 / _K)
    h = jnp.sum(p * jnp.log(p + 1e-10))
    perp_ref[...] = jnp.full((1, 1), jnp.exp(-h), jnp.float32)


_final_call = pl.pallas_call(
    _final_kernel,
    out_shape=(
        jax.ShapeDtypeStruct((_N, _D), jnp.float32),
        jax.ShapeDtypeStruct((1, 1), jnp.float32),
        jax.ShapeDtypeStruct((1, 1), jnp.float32),
    ),
)


def kernel(inputs, embedding):
    zf = inputs.reshape(_N, _D)
    idx = _argmin_call(zf, embedding).reshape(_N)
    q, counts = _sc_call(embedding, idx)
    qst, loss, perp = _final_call(zf, q, counts)
    return qst.reshape(inputs.shape), loss.reshape(()), perp.reshape(())
